# baseline (speedup 1.0000x reference)
import functools

import jax
import jax.numpy as jnp
from jax import lax
from jax.experimental import pallas as pl
from jax.experimental.pallas import tpu as pltpu
from jax.experimental.pallas import tpu_sc as plsc

N = 50000
M = 16
A = 64
B = 16
H = 128
DIN = 128
NG = 3
N0, K = 1000, 50
E = N * M

NW = 32
CHUNK = 128
CPW = 200
NCHUNK = NW * CPW
EPAD = NCHUNK * CHUNK


def _softplus(x):
    return jnp.maximum(x, 0.0) + jnp.log1p(jnp.exp(-jnp.abs(x)))


def _softplus_fast(x):
    return jnp.where(
        x >= 20.0, x, jnp.log(1.0 + jnp.exp(jnp.minimum(x, 20.0)))
    )


def _sigmoid(x):
    return 1.0 / (1.0 + jnp.exp(-x))


KB = 5
NGRP2 = CPW // (2 * KB)


def _sc_gather_body(table_hbm, idx_hbm, out_hbm, idx_v, rows_v,
                    gsemA, gsemB, osemA, osemB):
    wid = lax.axis_index("s") * 2 + lax.axis_index("c")
    base = wid * CPW
    pltpu.sync_copy(idx_hbm.at[pl.ds(base, CPW)], idx_v)

    def fire_g(g, half, sem):
        for b in range(KB):
            pltpu.async_copy(table_hbm.at[idx_v.at[g * KB + b]],
                             rows_v.at[half * KB + b], sem)

    def drain_g(g, half, sem):
        for b in range(KB):
            pltpu.make_async_copy(table_hbm.at[idx_v.at[g * KB + b]],
                                  rows_v.at[half * KB + b], sem).wait()

    def fire_o(g, half, sem):
        for b in range(KB):
            j = g * KB + b
            pltpu.async_copy(rows_v.at[half * KB + b],
                             out_hbm.at[pl.ds((base + j) * CHUNK, CHUNK)], sem)

    def drain_o(g, half, sem):
        for b in range(KB):
            j = g * KB + b
            pltpu.make_async_copy(rows_v.at[half * KB + b],
                                  out_hbm.at[pl.ds((base + j) * CHUNK, CHUNK)],
                                  sem).wait()

    fire_g(0, 0, gsemA)

    def step(i, carry):
        g = 2 * i
        drain_g(g, 0, gsemA)
        fire_o(g, 0, osemA)

        @pl.when(i > 0)
        def _():
            drain_o(g - 1, 1, osemB)

        fire_g(g + 1, 1, gsemB)
        drain_g(g + 1, 1, gsemB)
        fire_o(g + 1, 1, osemB)
        drain_o(g, 0, osemA)

        @pl.when(i < NGRP2 - 1)
        def _():
            fire_g(g + 2, 0, gsemA)

        return carry

    lax.fori_loop(0, NGRP2, step, 0)
    drain_o(2 * NGRP2 - 1, 1, osemB)


def _sc_gather(table, idx2d):
    mesh = plsc.VectorSubcoreMesh(core_axis_name="c", subcore_axis_name="s")
    f = pl.kernel(
        _sc_gather_body,
        out_type=jax.ShapeDtypeStruct((EPAD, A), jnp.float32),
        mesh=mesh,
        compiler_params=pltpu.CompilerParams(use_tc_tiling_on_sc=False),
        scratch_types=[
            pltpu.VMEM((CPW, CHUNK), jnp.int32),
            pltpu.VMEM((2 * KB, CHUNK, A), jnp.float32),
            pltpu.SemaphoreType.DMA,
            pltpu.SemaphoreType.DMA,
            pltpu.SemaphoreType.DMA,
            pltpu.SemaphoreType.DMA,
        ],
    )
    return f(table, idx2d)


def _embed(x, W, b):
    RB = 2000

    def body(x_ref, w_ref, b_ref, o_ref):
        o_ref[...] = (
            jnp.dot(x_ref[...], w_ref[...], preferred_element_type=jnp.float32)
            + b_ref[...]
        )

    return pl.pallas_call(
        body,
        grid=(N // RB,),
        in_specs=[
            pl.BlockSpec((RB, DIN), lambda i: (i, 0)),
            pl.BlockSpec((DIN, A), lambda i: (0, 0)),
            pl.BlockSpec((1, A), lambda i: (0, 0)),
        ],
        out_specs=pl.BlockSpec((RB, A), lambda i: (i, 0)),
        out_shape=jax.ShapeDtypeStruct((N, A), jnp.float32),
    )(x, W, b.reshape(1, A))


RB = 1000
EB = RB * M
GRID = N // RB


def _block_T(a_ref, g_ref, f_ref, ws, wn, wf, bias):
    Ts = jnp.dot(a_ref[...], ws, preferred_element_type=jnp.float32)
    T = jnp.dot(g_ref[...], wn, preferred_element_type=jnp.float32)
    T = T + jnp.dot(f_ref[...], wf, preferred_element_type=jnp.float32)
    return T.reshape(RB, M, H) + Ts[:, None, :] + bias[None]


def _p1_body(a_ref, g_ref, f_ref, ws_ref, wn_ref, wf_ref, b_ref, s_ref):
    i = pl.program_id(0)
    T3 = _block_T(a_ref, g_ref, f_ref, ws_ref[...], wn_ref[...], wf_ref[...],
                  b_ref[...])
    s = jnp.sum(T3, axis=(0, 1))
    Tc = T3 - (s * (1.0 / EB))[None, None, :]
    m2 = jnp.sum(Tc * Tc, axis=(0, 1))

    @pl.when(i == 0)
    def _():
        s_ref[...] = jnp.zeros_like(s_ref)

    s_ref[pl.ds(i, 1), :] = s[None]
    s_ref[56:57, :] += m2[None]


def _p1(atom, g, f2, Ws, Wn, Wf, bias):
    return pl.pallas_call(
        _p1_body,
        grid=(GRID,),
        in_specs=[
            pl.BlockSpec((RB, A), lambda i: (i, 0)),
            pl.BlockSpec((EB, A), lambda i: (i, 0)),
            pl.BlockSpec((EB, B), lambda i: (i, 0)),
            pl.BlockSpec((A, H), lambda i: (0, 0)),
            pl.BlockSpec((A, H), lambda i: (0, 0)),
            pl.BlockSpec((B, H), lambda i: (0, 0)),
            pl.BlockSpec((1, H), lambda i: (0, 0)),
        ],
        out_specs=pl.BlockSpec((64, H), lambda i: (0, 0)),
        out_shape=jax.ShapeDtypeStruct((64, H), jnp.float32),
    )(atom, g, f2, Ws, Wn, Wf, bias.reshape(1, H))


def _p2_body(a_ref, g_ref, f_ref, ws_ref, wn_ref, wf_ref, b_ref, s_ref,
             g2_ref, b2_ref, ns_ref, st_ref):
    i = pl.program_id(0)
    S = s_ref[...]
    sb = S[:GRID, :]
    mu = jnp.sum(sb, axis=0, keepdims=True) * (1.0 / E)
    d = sb * (1.0 / EB) - mu
    var = (S[56:57, :]
           + float(EB) * jnp.sum(d * d, axis=0, keepdims=True)) * (1.0 / E)
    scale = g2_ref[...] * lax.rsqrt(var + 1e-5)
    shift = b2_ref[...] - mu * scale
    wsS = ws_ref[...] * scale
    wnS = wn_ref[...] * scale
    wfS = wf_ref[...] * scale
    bS = b_ref[...] * scale + shift
    y = _block_T(a_ref, g_ref, f_ref, wsS, wnS, wfS, bS)
    filt = _sigmoid(y[..., :A])
    core = _softplus_fast(y[..., A:])
    ns = jnp.sum(filt * core, axis=1)
    ns_ref[...] = ns
    t = jnp.sum(ns, axis=0)
    nc = ns - (t * (1.0 / RB))[None, :]
    m2 = jnp.sum(nc * nc, axis=0)

    @pl.when(i == 0)
    def _():
        st_ref[...] = jnp.zeros_like(st_ref)

    st_ref[pl.ds(i, 1), :] = t[None]
    st_ref[56:57, :] += m2[None]


def _p2(atom, g, f2, Ws, Wn, Wf, bias, s, g2, b2):
    return pl.pallas_call(
        _p2_body,
        grid=(GRID,),
        in_specs=[
            pl.BlockSpec((RB, A), lambda i: (i, 0)),
            pl.BlockSpec((EB, A), lambda i: (i, 0)),
            pl.BlockSpec((EB, B), lambda i: (i, 0)),
            pl.BlockSpec((A, H), lambda i: (0, 0)),
            pl.BlockSpec((A, H), lambda i: (0, 0)),
            pl.BlockSpec((B, H), lambda i: (0, 0)),
            pl.BlockSpec((1, H), lambda i: (0, 0)),
            pl.BlockSpec((64, H), lambda i: (0, 0)),
            pl.BlockSpec((1, H), lambda i: (0, 0)),
            pl.BlockSpec((1, H), lambda i: (0, 0)),
        ],
        out_specs=[
            pl.BlockSpec((RB, A), lambda i: (i, 0)),
            pl.BlockSpec((64, A), lambda i: (0, 0)),
        ],
        out_shape=[
            jax.ShapeDtypeStruct((N, A), jnp.float32),
            jax.ShapeDtypeStruct((64, A), jnp.float32),
        ],
    )(atom, g, f2, Ws, Wn, Wf, bias.reshape(1, H), s,
      g2.reshape(1, H), b2.reshape(1, H))


def _p3_body(a_ref, ns_ref, st_ref, g1_ref, b1_ref, o_ref):
    S = st_ref[...]
    tb = S[:GRID, :]
    mu = jnp.sum(tb, axis=0, keepdims=True) * (1.0 / N)
    d = tb * (1.0 / RB) - mu
    var = (S[56:57, :]
           + float(RB) * jnp.sum(d * d, axis=0, keepdims=True)) * (1.0 / N)
    scale = g1_ref[...] * lax.rsqrt(var + 1e-5)
    shift = b1_ref[...] - mu * scale
    o_ref[...] = _softplus(a_ref[...] + ns_ref[...] * scale + shift)


def _p3(atom, ns, st, g1, b1):
    RB3 = 2000
    return pl.pallas_call(
        _p3_body,
        grid=(N // RB3,),
        in_specs=[
            pl.BlockSpec((RB3, A), lambda i: (i, 0)),
            pl.BlockSpec((RB3, A), lambda i: (i, 0)),
            pl.BlockSpec((64, A), lambda i: (0, 0)),
            pl.BlockSpec((1, A), lambda i: (0, 0)),
            pl.BlockSpec((1, A), lambda i: (0, 0)),
        ],
        out_specs=pl.BlockSpec((RB3, A), lambda i: (i, 0)),
        out_shape=jax.ShapeDtypeStruct((N, A), jnp.float32),
    )(atom, ns, st, g1.reshape(1, A), b1.reshape(1, A))


def _head_body(a_ref, fcw_ref, fcb_ref, ow_ref, ob_ref, o_ref):
    CB = a_ref.shape[0] // K
    a3 = a_ref[...].reshape(CB, K, A)
    mean = jnp.mean(a3, axis=1)
    cent = a3 - mean[:, None, :]
    var = jnp.sum(cent * cent, axis=1) * (1.0 / (K - 1))
    std = jnp.sqrt(var)
    crys = _softplus(jnp.concatenate([mean, std], axis=1))
    h = _softplus(
        jnp.dot(crys, fcw_ref[...], preferred_element_type=jnp.float32)
        + fcb_ref[...]
    )
    o_ref[...] = jnp.sum(h * ow_ref[...], axis=1, keepdims=True) + ob_ref[...]


def _head(atom, fc_W, fc_b, out_W, out_b):
    CB = 200

    return pl.pallas_call(
        _head_body,
        grid=(N0 // CB,),
        in_specs=[
            pl.BlockSpec((CB * K, A), lambda i: (i, 0)),
            pl.BlockSpec((H, H), lambda i: (0, 0)),
            pl.BlockSpec((1, H), lambda i: (0, 0)),
            pl.BlockSpec((1, H), lambda i: (0, 0)),
            pl.BlockSpec((1, 1), lambda i: (0, 0)),
        ],
        out_specs=pl.BlockSpec((CB, 1), lambda i: (i, 0)),
        out_shape=jax.ShapeDtypeStruct((N0, 1), jnp.float32),
    )(atom, fc_W, fc_b.reshape(1, H), out_W.reshape(1, H), out_b.reshape(1, 1))


def kernel(orig_atom_fea, nbr_fea, nbr_fea_idx, crystal_atom_idx,
           emb_W, emb_b, msg_W, msg_b, bn2_g, bn2_b, bn1_g, bn1_b,
           fc_W, fc_b, out_W, out_b):
    idx = nbr_fea_idx.reshape(-1).astype(jnp.int32)
    idx2d = jnp.concatenate(
        [idx, jnp.zeros((EPAD - E,), jnp.int32)]
    ).reshape(NCHUNK, CHUNK)
    f2 = nbr_fea.reshape(E, B)

    atom = _embed(orig_atom_fea, emb_W, emb_b)
    for i in range(NG):
        Wi = msg_W[i]
        Ws, Wn, Wf = Wi[:A], Wi[A:2 * A], Wi[2 * A:]
        g = _sc_gather(atom, idx2d)
        s = _p1(atom, g, f2, Ws, Wn, Wf, msg_b[i])
        ns, st = _p2(atom, g, f2, Ws, Wn, Wf, msg_b[i], s, bn2_g[i], bn2_b[i])
        atom = _p3(atom, ns, st, bn1_g[i], bn1_b[i])

    return _head(atom, fc_W, fc_b, out_W, out_b)
